# Initial kernel scaffold; baseline (speedup 1.0000x reference)
import jax
import jax.numpy as jnp
from jax.experimental import pallas as pl


def kernel(x, freqs_cis, w_gate, Wq, Wk, Wv, Wo, W1, W2):
    raise NotImplementedError("write your pallas kernel here")


# speedup vs baseline: 1.0178x; 1.0178x over previous
import functools

import jax
import jax.numpy as jnp
import numpy as np
from jax import lax
from jax.experimental import pallas as pl
from jax.experimental.pallas import tpu as pltpu
from jax.experimental.pallas import tpu_sc as plsc

B, S, D = 4, 2048, 768
NH, HD, DFF = 12, 64, 3072
K = S // 2
NEG = -1e9

NC, NS, L = 2, 16, 16
NW = NC * NS


def _route_body(xb_ref, wg_ref, gidx_ref, carr_ref, marr_ref, aux_ref,
                rank_scr, gcol_scr):
    b = pl.program_id(0)
    xb = xb_ref[0]
    wg = wg_ref[...]
    s_row = lax.dot_general(wg, xb, (((1,), (1,)), ((), ())),
                            preferred_element_type=jnp.float32)
    s_col = lax.transpose(s_row, (1, 0))

    CH = 256
    for c in range(S // CH):
        sc = s_col[c * CH:(c + 1) * CH, :]
        col_i = lax.broadcasted_iota(jnp.int32, (CH, S), 1)
        row_i = lax.broadcasted_iota(jnp.int32, (CH, S), 0) + c * CH
        gt = (s_row > sc).astype(jnp.float32)
        eqlt = ((s_row == sc) & (col_i < row_i)).astype(jnp.float32)
        rank_scr[c * CH:(c + 1) * CH, :] = (
            jnp.sum(gt, axis=1, keepdims=True)
            + jnp.sum(eqlt, axis=1, keepdims=True))

    rank_row = lax.transpose(rank_scr[...], (1, 0))
    sel = rank_row < float(K)
    marr_ref[0] = sel.astype(jnp.float32)
    carr_ref[0] = jnp.where(
        sel, rank_row.astype(jnp.int32) + b * K, 0).astype(jnp.int32)

    col_f = lax.broadcasted_iota(jnp.int32, (CH, S), 1).astype(jnp.float32)
    for c in range(K // CH):
        rv = (lax.broadcasted_iota(jnp.int32, (CH, 1), 0)
              .astype(jnp.float32) + c * CH)
        onehot = (rank_row == rv).astype(jnp.float32)
        gcol_scr[c * CH:(c + 1) * CH, :] = jnp.sum(
            onehot * col_f, axis=1, keepdims=True)
    gidx_ref[0] = (lax.transpose(gcol_scr[...], (1, 0))
                   + float(S) * b.astype(jnp.float32)).astype(jnp.int32)

    mb = jnp.mean(jax.nn.sigmoid(s_row), axis=1, keepdims=True)

    @pl.when(b == 0)
    def _():
        aux_ref[...] = jnp.zeros((1, 1), jnp.float32)

    aux_ref[...] += (mb - 0.5) ** 2 * (1.0 / B)


def _route(x, w_gate):
    return pl.pallas_call(
        _route_body,
        grid=(B,),
        in_specs=[
            pl.BlockSpec((1, S, D), lambda b: (b, 0, 0)),
            pl.BlockSpec((1, D), lambda b: (0, 0)),
        ],
        out_specs=[
            pl.BlockSpec((1, 1, K), lambda b: (b, 0, 0)),
            pl.BlockSpec((1, 1, S), lambda b: (b, 0, 0)),
            pl.BlockSpec((1, 1, S), lambda b: (b, 0, 0)),
            pl.BlockSpec((1, 1), lambda b: (0, 0)),
        ],
        out_shape=[
            jax.ShapeDtypeStruct((B, 1, K), jnp.int32),
            jax.ShapeDtypeStruct((B, 1, S), jnp.int32),
            jax.ShapeDtypeStruct((B, 1, S), jnp.float32),
            jax.ShapeDtypeStruct((1, 1), jnp.float32),
        ],
        scratch_shapes=[
            pltpu.VMEM((S, 1), jnp.float32),
            pltpu.VMEM((K, 1), jnp.float32),
        ],
    )(x, w_gate)


_ROWS_G = (B * K) // NW


@functools.cache
def _sc_mesh():
    return plsc.VectorSubcoreMesh(core_axis_name="c", subcore_axis_name="s",
                                  num_cores=NC, num_subcores=NS)


@functools.cache
def _gather_kernel():
    @functools.partial(
        pl.kernel,
        out_type=jax.ShapeDtypeStruct((B * K, D), jnp.float32),
        mesh=_sc_mesh(),
        compiler_params=pltpu.CompilerParams(needs_layout_passes=False),
        scratch_types=[
            pltpu.VMEM((_ROWS_G,), jnp.int32),
            pltpu.VMEM((_ROWS_G, D), jnp.float32),
            pltpu.SemaphoreType.DMA,
        ],
    )
    def gather(xf_hbm, gidx_hbm, sel_hbm, idx_v, rows_v, sem):
        wid = lax.axis_index("s") * NC + lax.axis_index("c")
        base = wid * _ROWS_G
        pltpu.sync_copy(gidx_hbm.at[pl.ds(base, _ROWS_G)], idx_v)
        pltpu.async_copy(xf_hbm.at[idx_v], rows_v, sem).wait()
        pltpu.sync_copy(rows_v, sel_hbm.at[pl.ds(base, _ROWS_G)])

    return gather


def _gather(xf, gidx):
    return _gather_kernel()(xf, gidx)


def _attn_body(sel_ref, th_ref, wq_ref, wk_ref, wv_ref, wo_ref, out_ref):
    h = sel_ref[0]
    th = th_ref[...]
    cos = jnp.cos(th)
    sin = jnp.sin(th)
    hn = h * lax.rsqrt(jnp.mean(h * h, axis=-1, keepdims=True) + 1e-6)
    q = jnp.dot(hn, wq_ref[...], preferred_element_type=jnp.float32)
    k = jnp.dot(hn, wk_ref[...], preferred_element_type=jnp.float32)
    v = jnp.dot(hn, wv_ref[...], preferred_element_type=jnp.float32)
    mask = (lax.broadcasted_iota(jnp.int32, (K, K), 0)
            >= lax.broadcasted_iota(jnp.int32, (K, K), 1))
    outs = []
    for hd in range(NH):
        o = hd * HD
        q1, q2 = q[:, o:o + 32], q[:, o + 32:o + 64]
        k1, k2 = k[:, o:o + 32], k[:, o + 32:o + 64]
        rq = jnp.concatenate([q1 * cos - q2 * sin, q2 * cos + q1 * sin], 1)
        rk = jnp.concatenate([k1 * cos - k2 * sin, k2 * cos + k1 * sin], 1)
        logits = lax.dot_general(rq, rk, (((1,), (1,)), ((), ())),
                                 preferred_element_type=jnp.float32) * 0.125
        logits = jnp.where(mask, logits, NEG)
        m = jnp.max(logits, axis=1, keepdims=True)
        p = jnp.exp(logits - m)
        p = p / jnp.sum(p, axis=1, keepdims=True)
        outs.append(jnp.dot(p, v[:, o:o + HD],
                            preferred_element_type=jnp.float32))
    o_all = jnp.concatenate(outs, axis=1)
    out_ref[0] = h + jnp.dot(o_all, wo_ref[...],
                             preferred_element_type=jnp.float32)


def _attn(selb, theta, wq, wk, wv, wo):
    return pl.pallas_call(
        _attn_body,
        grid=(B,),
        in_specs=[
            pl.BlockSpec((1, K, D), lambda b: (b, 0, 0)),
            pl.BlockSpec((K, HD // 2), lambda b: (0, 0)),
            pl.BlockSpec((D, D), lambda b: (0, 0)),
            pl.BlockSpec((D, D), lambda b: (0, 0)),
            pl.BlockSpec((D, D), lambda b: (0, 0)),
            pl.BlockSpec((D, D), lambda b: (0, 0)),
        ],
        out_specs=pl.BlockSpec((1, K, D), lambda b: (b, 0, 0)),
        out_shape=jax.ShapeDtypeStruct((B, K, D), jnp.float32),
    )(selb, theta, wq, wk, wv, wo)


_RCH = 256


def _mlp_body(h_ref, w1_ref, w2_ref, out_ref):
    h = h_ref[0]
    hn = h * lax.rsqrt(jnp.mean(h * h, axis=-1, keepdims=True) + 1e-6)
    a = jax.nn.gelu(jnp.dot(hn, w1_ref[...],
                            preferred_element_type=jnp.float32))
    out_ref[0] = h + jnp.dot(a, w2_ref[...],
                             preferred_element_type=jnp.float32)


def _mlp(h, w1, w2):
    return pl.pallas_call(
        _mlp_body,
        grid=(B, K // _RCH),
        in_specs=[
            pl.BlockSpec((1, _RCH, D), lambda b, r: (b, r, 0)),
            pl.BlockSpec((D, DFF), lambda b, r: (0, 0)),
            pl.BlockSpec((DFF, D), lambda b, r: (0, 0)),
        ],
        out_specs=pl.BlockSpec((1, _RCH, D), lambda b, r: (b, r, 0)),
        out_shape=jax.ShapeDtypeStruct((B, K, D), jnp.float32),
    )(h, w1, w2)


_ROWS_C = 64
_NCH = (B * S) // (NW * _ROWS_C)


@functools.cache
def _combine_kernel():
    @functools.partial(
        pl.kernel,
        out_type=jax.ShapeDtypeStruct((B * S, D), jnp.float32),
        mesh=_sc_mesh(),
        compiler_params=pltpu.CompilerParams(needs_layout_passes=False),
        scratch_types=[
            pltpu.VMEM((_ROWS_C,), jnp.int32),
            pltpu.VMEM((_ROWS_C,), jnp.float32),
            pltpu.VMEM((_ROWS_C, D), jnp.float32),
            pltpu.VMEM((_ROWS_C, D), jnp.float32),
            pltpu.SemaphoreType.DMA,
        ],
    )
    def combine(xf_hbm, bf_hbm, carr_hbm, marr_hbm, out_hbm,
                idx_v, m_v, bufx, bufb, sem):
        wid = lax.axis_index("s") * NC + lax.axis_index("c")

        def chunk(ci, carry):
            base = wid * (_NCH * _ROWS_C) + ci * _ROWS_C
            pltpu.sync_copy(carr_hbm.at[pl.ds(base, _ROWS_C)], idx_v)
            pltpu.sync_copy(marr_hbm.at[pl.ds(base, _ROWS_C)], m_v)
            pltpu.sync_copy(xf_hbm.at[pl.ds(base, _ROWS_C)], bufx)
            pltpu.async_copy(bf_hbm.at[idx_v], bufb, sem).wait()

            def row(j, c2):
                mj = plsc.load_gather(
                    m_v, [jnp.full((L,), j, dtype=jnp.int32)])
                for t in range(D // L):
                    xv = bufx[j, pl.ds(t * L, L)]
                    bv = bufb[j, pl.ds(t * L, L)]
                    bufx[j, pl.ds(t * L, L)] = xv + mj * (bv - xv)
                return c2

            lax.fori_loop(0, _ROWS_C, row, 0)
            pltpu.sync_copy(bufx, out_hbm.at[pl.ds(base, _ROWS_C)])
            return carry

        lax.fori_loop(0, _NCH, chunk, 0)

    return combine


def _combine(xf, bf, carr, marr):
    return _combine_kernel()(xf, bf, carr, marr)


def _rope_perm():
    one = np.concatenate([np.arange(0, HD, 2), np.arange(1, HD, 2)])
    return np.concatenate([one + HD * h for h in range(NH)])


_PERM = _rope_perm()


def kernel(x, freqs_cis, w_gate, Wq, Wk, Wv, Wo, W1, W2):
    wg2 = w_gate.reshape(1, D)
    gidx, carr, marr, aux = _route(x, wg2)

    x_flat = x.reshape(B * S, D)
    selected = _gather(x_flat, gidx.reshape(B * K))
    selb = selected.reshape(B, K, D)

    h1 = _attn(selb, freqs_cis[:K], Wq[:, _PERM], Wk[:, _PERM], Wv, Wo)
    block_out = _mlp(h1, W1, W2)

    out = _combine(x_flat, block_out.reshape(B * K, D),
                   carr.reshape(B * S), marr.reshape(B * S))
    return out.reshape(B, S, D), aux[0, 0]


# speedup vs baseline: 1.5182x; 1.4917x over previous
import functools

import jax
import jax.numpy as jnp
import numpy as np
from jax import lax
from jax.experimental import pallas as pl
from jax.experimental.pallas import tpu as pltpu
from jax.experimental.pallas import tpu_sc as plsc

B, S, D = 4, 2048, 768
NH, HD, DFF = 12, 64, 3072
K = S // 2
NEG = -1e9

NC, NS, L = 2, 16, 16
NW = NC * NS


def _route_body(xb_ref, wg_ref, gidx_ref, uidx_ref, aux_ref,
                rank_scr, gcol_scr):
    b = pl.program_id(0)
    xb = xb_ref[0]
    wg = wg_ref[...]
    s_row = lax.dot_general(wg, xb, (((1,), (1,)), ((), ())),
                            preferred_element_type=jnp.float32)
    s_col = lax.transpose(s_row, (1, 0))

    CH = 256
    for c in range(S // CH):
        sc = s_col[c * CH:(c + 1) * CH, :]
        col_i = lax.broadcasted_iota(jnp.int32, (CH, S), 1)
        row_i = lax.broadcasted_iota(jnp.int32, (CH, S), 0) + c * CH
        gt = (s_row > sc).astype(jnp.float32)
        eqlt = ((s_row == sc) & (col_i < row_i)).astype(jnp.float32)
        rank_scr[c * CH:(c + 1) * CH, :] = (
            jnp.sum(gt, axis=1, keepdims=True)
            + jnp.sum(eqlt, axis=1, keepdims=True))

    rank_row = lax.transpose(rank_scr[...], (1, 0))

    col_f = lax.broadcasted_iota(jnp.int32, (CH, S), 1).astype(jnp.float32)
    for c in range(S // CH):
        rv = (lax.broadcasted_iota(jnp.int32, (CH, 1), 0)
              .astype(jnp.float32) + c * CH)
        onehot = (rank_row == rv).astype(jnp.float32)
        gcol_scr[c * CH:(c + 1) * CH, :] = jnp.sum(
            onehot * col_f, axis=1, keepdims=True)
    inv_row = (lax.transpose(gcol_scr[...], (1, 0))
               + float(S) * b.astype(jnp.float32))
    gidx_ref[0] = inv_row[:, :K].astype(jnp.int32)
    uidx_ref[0] = inv_row[:, K:].astype(jnp.int32)

    mb = jnp.mean(jax.nn.sigmoid(s_row), axis=1, keepdims=True)

    @pl.when(b == 0)
    def _():
        aux_ref[...] = jnp.zeros((1, 1), jnp.float32)

    aux_ref[...] += (mb - 0.5) ** 2 * (1.0 / B)


def _route(x, w_gate):
    return pl.pallas_call(
        _route_body,
        grid=(B,),
        in_specs=[
            pl.BlockSpec((1, S, D), lambda b: (b, 0, 0)),
            pl.BlockSpec((1, D), lambda b: (0, 0)),
        ],
        out_specs=[
            pl.BlockSpec((1, 1, K), lambda b: (b, 0, 0)),
            pl.BlockSpec((1, 1, K), lambda b: (b, 0, 0)),
            pl.BlockSpec((1, 1), lambda b: (0, 0)),
        ],
        out_shape=[
            jax.ShapeDtypeStruct((B, 1, K), jnp.int32),
            jax.ShapeDtypeStruct((B, 1, K), jnp.int32),
            jax.ShapeDtypeStruct((1, 1), jnp.float32),
        ],
        scratch_shapes=[
            pltpu.VMEM((S, 1), jnp.float32),
            pltpu.VMEM((S, 1), jnp.float32),
        ],
    )(x, w_gate)


_ROWS_G = (B * K) // NW


@functools.cache
def _sc_mesh():
    return plsc.VectorSubcoreMesh(core_axis_name="c", subcore_axis_name="s",
                                  num_cores=NC, num_subcores=NS)


@functools.cache
def _gather_kernel():
    @functools.partial(
        pl.kernel,
        out_type=jax.ShapeDtypeStruct((B * K, D), jnp.float32),
        mesh=_sc_mesh(),
        compiler_params=pltpu.CompilerParams(needs_layout_passes=False),
        scratch_types=[
            pltpu.VMEM((_ROWS_G,), jnp.int32),
            pltpu.VMEM((_ROWS_G, D), jnp.float32),
            pltpu.SemaphoreType.DMA,
        ],
    )
    def gather(xf_hbm, gidx_hbm, sel_hbm, idx_v, rows_v, sem):
        wid = lax.axis_index("s") * NC + lax.axis_index("c")
        base = wid * _ROWS_G
        pltpu.sync_copy(gidx_hbm.at[pl.ds(base, _ROWS_G)], idx_v)
        pltpu.async_copy(xf_hbm.at[idx_v], rows_v, sem).wait()
        pltpu.sync_copy(rows_v, sel_hbm.at[pl.ds(base, _ROWS_G)])

    return gather


def _gather(xf, gidx):
    return _gather_kernel()(xf, gidx)


def _attn_body(sel_ref, th_ref, wq_ref, wk_ref, wv_ref, wo_ref, out_ref):
    h = sel_ref[0]
    th = th_ref[...]
    cos = jnp.cos(th)
    sin = jnp.sin(th)
    hn = h * lax.rsqrt(jnp.mean(h * h, axis=-1, keepdims=True) + 1e-6)
    q = jnp.dot(hn, wq_ref[...], preferred_element_type=jnp.float32)
    k = jnp.dot(hn, wk_ref[...], preferred_element_type=jnp.float32)
    v = jnp.dot(hn, wv_ref[...], preferred_element_type=jnp.float32)
    mask = (lax.broadcasted_iota(jnp.int32, (K, K), 0)
            >= lax.broadcasted_iota(jnp.int32, (K, K), 1))
    outs = []
    for hd in range(NH):
        o = hd * HD
        q1, q2 = q[:, o:o + 32], q[:, o + 32:o + 64]
        k1, k2 = k[:, o:o + 32], k[:, o + 32:o + 64]
        rq = jnp.concatenate([q1 * cos - q2 * sin, q2 * cos + q1 * sin], 1)
        rk = jnp.concatenate([k1 * cos - k2 * sin, k2 * cos + k1 * sin], 1)
        logits = lax.dot_general(rq, rk, (((1,), (1,)), ((), ())),
                                 preferred_element_type=jnp.float32) * 0.125
        logits = jnp.where(mask, logits, NEG)
        m = jnp.max(logits, axis=1, keepdims=True)
        p = jnp.exp(logits - m)
        p = p / jnp.sum(p, axis=1, keepdims=True)
        outs.append(jnp.dot(p, v[:, o:o + HD],
                            preferred_element_type=jnp.float32))
    o_all = jnp.concatenate(outs, axis=1)
    out_ref[0] = h + jnp.dot(o_all, wo_ref[...],
                             preferred_element_type=jnp.float32)


def _attn(selb, theta, wq, wk, wv, wo):
    return pl.pallas_call(
        _attn_body,
        grid=(B,),
        in_specs=[
            pl.BlockSpec((1, K, D), lambda b: (b, 0, 0)),
            pl.BlockSpec((K, HD // 2), lambda b: (0, 0)),
            pl.BlockSpec((D, D), lambda b: (0, 0)),
            pl.BlockSpec((D, D), lambda b: (0, 0)),
            pl.BlockSpec((D, D), lambda b: (0, 0)),
            pl.BlockSpec((D, D), lambda b: (0, 0)),
        ],
        out_specs=pl.BlockSpec((1, K, D), lambda b: (b, 0, 0)),
        out_shape=jax.ShapeDtypeStruct((B, K, D), jnp.float32),
    )(selb, theta, wq, wk, wv, wo)


_RCH = 256


def _mlp_body(h_ref, w1_ref, w2_ref, out_ref):
    h = h_ref[0]
    hn = h * lax.rsqrt(jnp.mean(h * h, axis=-1, keepdims=True) + 1e-6)
    a = jax.nn.gelu(jnp.dot(hn, w1_ref[...],
                            preferred_element_type=jnp.float32))
    out_ref[0] = h + jnp.dot(a, w2_ref[...],
                             preferred_element_type=jnp.float32)


def _mlp(h, w1, w2):
    return pl.pallas_call(
        _mlp_body,
        grid=(B, K // _RCH),
        in_specs=[
            pl.BlockSpec((1, _RCH, D), lambda b, r: (b, r, 0)),
            pl.BlockSpec((D, DFF), lambda b, r: (0, 0)),
            pl.BlockSpec((DFF, D), lambda b, r: (0, 0)),
        ],
        out_specs=pl.BlockSpec((1, _RCH, D), lambda b, r: (b, r, 0)),
        out_shape=jax.ShapeDtypeStruct((B, K, D), jnp.float32),
    )(h, w1, w2)


@functools.cache
def _combine_kernel():
    @functools.partial(
        pl.kernel,
        out_type=jax.ShapeDtypeStruct((B * S, D), jnp.float32),
        mesh=_sc_mesh(),
        compiler_params=pltpu.CompilerParams(needs_layout_passes=False),
        scratch_types=[
            pltpu.VMEM((_ROWS_G,), jnp.int32),
            pltpu.VMEM((_ROWS_G, D), jnp.float32),
            pltpu.SemaphoreType.DMA,
        ],
    )
    def combine(xf_hbm, bf_hbm, gidx_hbm, uidx_hbm, out_hbm,
                idx_v, buf, sem):
        wid = lax.axis_index("s") * NC + lax.axis_index("c")
        base = wid * _ROWS_G
        pltpu.sync_copy(uidx_hbm.at[pl.ds(base, _ROWS_G)], idx_v)
        pltpu.async_copy(xf_hbm.at[idx_v], buf, sem).wait()
        pltpu.async_copy(buf, out_hbm.at[idx_v], sem).wait()
        pltpu.sync_copy(bf_hbm.at[pl.ds(base, _ROWS_G)], buf)
        pltpu.sync_copy(gidx_hbm.at[pl.ds(base, _ROWS_G)], idx_v)
        pltpu.async_copy(buf, out_hbm.at[idx_v], sem).wait()

    return combine


def _combine(xf, bf, gidx, uidx):
    return _combine_kernel()(xf, bf, gidx, uidx)


def _rope_perm():
    one = np.concatenate([np.arange(0, HD, 2), np.arange(1, HD, 2)])
    return np.concatenate([one + HD * h for h in range(NH)])


_PERM = _rope_perm()


def kernel(x, freqs_cis, w_gate, Wq, Wk, Wv, Wo, W1, W2):
    wg2 = w_gate.reshape(1, D)
    gidx, uidx, aux = _route(x, wg2)

    x_flat = x.reshape(B * S, D)
    gidx_f = gidx.reshape(B * K)
    selected = _gather(x_flat, gidx_f)
    selb = selected.reshape(B, K, D)

    h1 = _attn(selb, freqs_cis[:K], Wq[:, _PERM], Wk[:, _PERM], Wv, Wo)
    block_out = _mlp(h1, W1, W2)

    out = _combine(x_flat, block_out.reshape(B * K, D),
                   gidx_f, uidx.reshape(B * K))
    return out.reshape(B, S, D), aux[0, 0]


# speedup vs baseline: 1.7306x; 1.1399x over previous
import functools

import jax
import jax.numpy as jnp
import numpy as np
from jax import lax
from jax.experimental import pallas as pl
from jax.experimental.pallas import tpu as pltpu
from jax.experimental.pallas import tpu_sc as plsc

B, S, D = 4, 2048, 768
NH, HD, DFF = 12, 64, 3072
K = S // 2
NEG = -1e9

NC, NS, L = 2, 16, 16
NW = NC * NS


def _route_body(xb_ref, wg_ref, gidx_ref, uidx_ref, aux_ref,
                rank_scr, gcol_scr):
    b = pl.program_id(0)
    xb = xb_ref[0]
    wg = wg_ref[...]
    s_row = lax.dot_general(wg, xb, (((1,), (1,)), ((), ())),
                            preferred_element_type=jnp.float32)
    s_col = lax.transpose(s_row, (1, 0))

    CH = 256
    for c in range(S // CH):
        sc = s_col[c * CH:(c + 1) * CH, :]
        col_i = lax.broadcasted_iota(jnp.int32, (CH, S), 1)
        row_i = lax.broadcasted_iota(jnp.int32, (CH, S), 0) + c * CH
        gt = (s_row > sc).astype(jnp.float32)
        eqlt = ((s_row == sc) & (col_i < row_i)).astype(jnp.float32)
        rank_scr[c * CH:(c + 1) * CH, :] = (
            jnp.sum(gt, axis=1, keepdims=True)
            + jnp.sum(eqlt, axis=1, keepdims=True))

    rank_row = lax.transpose(rank_scr[...], (1, 0))

    col_f = lax.broadcasted_iota(jnp.int32, (CH, S), 1).astype(jnp.float32)
    for c in range(S // CH):
        rv = (lax.broadcasted_iota(jnp.int32, (CH, 1), 0)
              .astype(jnp.float32) + c * CH)
        onehot = (rank_row == rv).astype(jnp.float32)
        gcol_scr[c * CH:(c + 1) * CH, :] = jnp.sum(
            onehot * col_f, axis=1, keepdims=True)
    inv_row = (lax.transpose(gcol_scr[...], (1, 0))
               + float(S) * b.astype(jnp.float32))
    gidx_ref[0] = inv_row[:, :K].astype(jnp.int32)
    uidx_ref[0] = inv_row[:, K:].astype(jnp.int32)

    mb = jnp.mean(jax.nn.sigmoid(s_row), axis=1, keepdims=True)

    @pl.when(b == 0)
    def _():
        aux_ref[...] = jnp.zeros((1, 1), jnp.float32)

    aux_ref[...] += (mb - 0.5) ** 2 * (1.0 / B)


def _route(x, w_gate):
    return pl.pallas_call(
        _route_body,
        grid=(B,),
        in_specs=[
            pl.BlockSpec((1, S, D), lambda b: (b, 0, 0)),
            pl.BlockSpec((1, D), lambda b: (0, 0)),
        ],
        out_specs=[
            pl.BlockSpec((1, 1, K), lambda b: (b, 0, 0)),
            pl.BlockSpec((1, 1, K), lambda b: (b, 0, 0)),
            pl.BlockSpec((1, 1), lambda b: (0, 0)),
        ],
        out_shape=[
            jax.ShapeDtypeStruct((B, 1, K), jnp.int32),
            jax.ShapeDtypeStruct((B, 1, K), jnp.int32),
            jax.ShapeDtypeStruct((1, 1), jnp.float32),
        ],
        scratch_shapes=[
            pltpu.VMEM((S, 1), jnp.float32),
            pltpu.VMEM((S, 1), jnp.float32),
        ],
    )(x, w_gate)


_ROWS_G = (B * K) // NW


@functools.cache
def _sc_mesh():
    return plsc.VectorSubcoreMesh(core_axis_name="c", subcore_axis_name="s",
                                  num_cores=NC, num_subcores=NS)


@functools.cache
def _gather_kernel():
    @functools.partial(
        pl.kernel,
        out_type=jax.ShapeDtypeStruct((B * K, D), jnp.float32),
        mesh=_sc_mesh(),
        compiler_params=pltpu.CompilerParams(needs_layout_passes=False),
        scratch_types=[
            pltpu.VMEM((_ROWS_G,), jnp.int32),
            pltpu.VMEM((_ROWS_G, D), jnp.float32),
            pltpu.SemaphoreType.DMA,
        ],
    )
    def gather(xf_hbm, gidx_hbm, sel_hbm, idx_v, rows_v, sem):
        wid = lax.axis_index("s") * NC + lax.axis_index("c")
        base = wid * _ROWS_G
        pltpu.sync_copy(gidx_hbm.at[pl.ds(base, _ROWS_G)], idx_v)
        pltpu.async_copy(xf_hbm.at[idx_v], rows_v, sem).wait()
        pltpu.sync_copy(rows_v, sel_hbm.at[pl.ds(base, _ROWS_G)])

    return gather


def _gather(xf, gidx):
    return _gather_kernel()(xf, gidx)


def _attn_body(sel_ref, th_ref, wq_ref, wk_ref, wv_ref, wo_ref, out_ref):
    h = sel_ref[0]
    th = th_ref[...]
    cos = jnp.cos(th)
    sin = jnp.sin(th)
    hn = (h * lax.rsqrt(jnp.mean(h * h, axis=-1, keepdims=True) + 1e-6)
          ).astype(jnp.bfloat16)
    q = jnp.dot(hn, wq_ref[...].astype(jnp.bfloat16),
                preferred_element_type=jnp.float32)
    k = jnp.dot(hn, wk_ref[...].astype(jnp.bfloat16),
                preferred_element_type=jnp.float32)
    v = jnp.dot(hn, wv_ref[...].astype(jnp.bfloat16),
                preferred_element_type=jnp.float32).astype(jnp.bfloat16)
    mask = (lax.broadcasted_iota(jnp.int32, (K, K), 0)
            >= lax.broadcasted_iota(jnp.int32, (K, K), 1))
    outs = []
    for hd in range(NH):
        o = hd * HD
        q1, q2 = q[:, o:o + 32], q[:, o + 32:o + 64]
        k1, k2 = k[:, o:o + 32], k[:, o + 32:o + 64]
        rq = jnp.concatenate([q1 * cos - q2 * sin, q2 * cos + q1 * sin],
                             1).astype(jnp.bfloat16)
        rk = jnp.concatenate([k1 * cos - k2 * sin, k2 * cos + k1 * sin],
                             1).astype(jnp.bfloat16)
        logits = lax.dot_general(rq, rk, (((1,), (1,)), ((), ())),
                                 preferred_element_type=jnp.float32) * 0.125
        logits = jnp.where(mask, logits, NEG)
        m = jnp.max(logits, axis=1, keepdims=True)
        p = jnp.exp(logits - m)
        p = (p / jnp.sum(p, axis=1, keepdims=True)).astype(jnp.bfloat16)
        outs.append(lax.dot_general(p, v[:, o:o + HD], (((1,), (0,)), ((), ())),
                                    preferred_element_type=jnp.float32))
    o_all = jnp.concatenate(outs, axis=1).astype(jnp.bfloat16)
    out_ref[0] = h + jnp.dot(o_all, wo_ref[...].astype(jnp.bfloat16),
                             preferred_element_type=jnp.float32)


def _attn(selb, theta, wq, wk, wv, wo):
    return pl.pallas_call(
        _attn_body,
        grid=(B,),
        in_specs=[
            pl.BlockSpec((1, K, D), lambda b: (b, 0, 0)),
            pl.BlockSpec((K, HD // 2), lambda b: (0, 0)),
            pl.BlockSpec((D, D), lambda b: (0, 0)),
            pl.BlockSpec((D, D), lambda b: (0, 0)),
            pl.BlockSpec((D, D), lambda b: (0, 0)),
            pl.BlockSpec((D, D), lambda b: (0, 0)),
        ],
        out_specs=pl.BlockSpec((1, K, D), lambda b: (b, 0, 0)),
        out_shape=jax.ShapeDtypeStruct((B, K, D), jnp.float32),
    )(selb, theta, wq, wk, wv, wo)


_RCH = 256


def _mlp_body(h_ref, w1_ref, w2_ref, out_ref):
    h = h_ref[0]
    hn = (h * lax.rsqrt(jnp.mean(h * h, axis=-1, keepdims=True) + 1e-6)
          ).astype(jnp.bfloat16)
    a = jax.nn.gelu(jnp.dot(hn, w1_ref[...].astype(jnp.bfloat16),
                            preferred_element_type=jnp.float32))
    out_ref[0] = h + jnp.dot(a.astype(jnp.bfloat16),
                             w2_ref[...].astype(jnp.bfloat16),
                             preferred_element_type=jnp.float32)


def _mlp(h, w1, w2):
    return pl.pallas_call(
        _mlp_body,
        grid=(B, K // _RCH),
        in_specs=[
            pl.BlockSpec((1, _RCH, D), lambda b, r: (b, r, 0)),
            pl.BlockSpec((D, DFF), lambda b, r: (0, 0)),
            pl.BlockSpec((DFF, D), lambda b, r: (0, 0)),
        ],
        out_specs=pl.BlockSpec((1, _RCH, D), lambda b, r: (b, r, 0)),
        out_shape=jax.ShapeDtypeStruct((B, K, D), jnp.float32),
    )(h, w1, w2)


@functools.cache
def _combine_kernel():
    @functools.partial(
        pl.kernel,
        out_type=jax.ShapeDtypeStruct((B * S, D), jnp.float32),
        mesh=_sc_mesh(),
        compiler_params=pltpu.CompilerParams(needs_layout_passes=False),
        scratch_types=[
            pltpu.VMEM((_ROWS_G,), jnp.int32),
            pltpu.VMEM((_ROWS_G, D), jnp.float32),
            pltpu.SemaphoreType.DMA,
        ],
    )
    def combine(xf_hbm, bf_hbm, gidx_hbm, uidx_hbm, out_hbm,
                idx_v, buf, sem):
        wid = lax.axis_index("s") * NC + lax.axis_index("c")
        base = wid * _ROWS_G
        pltpu.sync_copy(uidx_hbm.at[pl.ds(base, _ROWS_G)], idx_v)
        pltpu.async_copy(xf_hbm.at[idx_v], buf, sem).wait()
        pltpu.async_copy(buf, out_hbm.at[idx_v], sem).wait()
        pltpu.sync_copy(bf_hbm.at[pl.ds(base, _ROWS_G)], buf)
        pltpu.sync_copy(gidx_hbm.at[pl.ds(base, _ROWS_G)], idx_v)
        pltpu.async_copy(buf, out_hbm.at[idx_v], sem).wait()

    return combine


def _combine(xf, bf, gidx, uidx):
    return _combine_kernel()(xf, bf, gidx, uidx)


def _rope_perm():
    one = np.concatenate([np.arange(0, HD, 2), np.arange(1, HD, 2)])
    return np.concatenate([one + HD * h for h in range(NH)])


_PERM = _rope_perm()


def kernel(x, freqs_cis, w_gate, Wq, Wk, Wv, Wo, W1, W2):
    wg2 = w_gate.reshape(1, D)
    gidx, uidx, aux = _route(x, wg2)

    x_flat = x.reshape(B * S, D)
    gidx_f = gidx.reshape(B * K)
    selected = _gather(x_flat, gidx_f)
    selb = selected.reshape(B, K, D)

    h1 = _attn(selb, freqs_cis[:K], Wq[:, _PERM], Wk[:, _PERM], Wv, Wo)
    block_out = _mlp(h1, W1, W2)

    out = _combine(x_flat, block_out.reshape(B * K, D),
                   gidx_f, uidx.reshape(B * K))
    return out.reshape(B, S, D), aux[0, 0]


# speedup vs baseline: 2.5769x; 1.4890x over previous
import functools

import jax
import jax.numpy as jnp
import numpy as np
from jax import lax
from jax.experimental import pallas as pl
from jax.experimental.pallas import tpu as pltpu
from jax.experimental.pallas import tpu_sc as plsc

B, S, D = 4, 2048, 768
NH, HD, DFF = 12, 64, 3072
K = S // 2
NEG = -1e9

NC, NS, L = 2, 16, 16
NW = NC * NS


def _route_body(xb_ref, wg_ref, gidx_ref, uidx_ref, aux_ref,
                rank_scr, gcol_scr):
    b = pl.program_id(0)
    xb = xb_ref[0]
    wg = wg_ref[...]
    s_row = lax.dot_general(wg, xb, (((1,), (1,)), ((), ())),
                            preferred_element_type=jnp.float32)
    s_col = lax.transpose(s_row, (1, 0))

    CH = 256
    for c in range(S // CH):
        sc = s_col[c * CH:(c + 1) * CH, :]
        col_i = lax.broadcasted_iota(jnp.int32, (CH, S), 1)
        row_i = lax.broadcasted_iota(jnp.int32, (CH, S), 0) + c * CH
        gt = (s_row > sc).astype(jnp.float32)
        eqlt = ((s_row == sc) & (col_i < row_i)).astype(jnp.float32)
        rank_scr[c * CH:(c + 1) * CH, :] = (
            jnp.sum(gt, axis=1, keepdims=True)
            + jnp.sum(eqlt, axis=1, keepdims=True))

    rank_row = lax.transpose(rank_scr[...], (1, 0))

    col_f = lax.broadcasted_iota(jnp.int32, (CH, S), 1).astype(jnp.float32)
    for c in range(S // CH):
        rv = (lax.broadcasted_iota(jnp.int32, (CH, 1), 0)
              .astype(jnp.float32) + c * CH)
        onehot = (rank_row == rv).astype(jnp.float32)
        gcol_scr[c * CH:(c + 1) * CH, :] = jnp.sum(
            onehot * col_f, axis=1, keepdims=True)
    inv_row = (lax.transpose(gcol_scr[...], (1, 0))
               + float(S) * b.astype(jnp.float32))
    gidx_ref[0] = inv_row[:, :K].astype(jnp.int32)
    uidx_ref[0] = inv_row[:, K:].astype(jnp.int32)

    mb = jnp.mean(jax.nn.sigmoid(s_row), axis=1, keepdims=True)

    @pl.when(b == 0)
    def _():
        aux_ref[...] = jnp.zeros((1, 1), jnp.float32)

    aux_ref[...] += (mb - 0.5) ** 2 * (1.0 / B)


def _route(x, w_gate):
    return pl.pallas_call(
        _route_body,
        grid=(B,),
        in_specs=[
            pl.BlockSpec((1, S, D), lambda b: (b, 0, 0)),
            pl.BlockSpec((1, D), lambda b: (0, 0)),
        ],
        out_specs=[
            pl.BlockSpec((1, 1, K), lambda b: (b, 0, 0)),
            pl.BlockSpec((1, 1, K), lambda b: (b, 0, 0)),
            pl.BlockSpec((1, 1), lambda b: (0, 0)),
        ],
        out_shape=[
            jax.ShapeDtypeStruct((B, 1, K), jnp.int32),
            jax.ShapeDtypeStruct((B, 1, K), jnp.int32),
            jax.ShapeDtypeStruct((1, 1), jnp.float32),
        ],
        scratch_shapes=[
            pltpu.VMEM((S, 1), jnp.float32),
            pltpu.VMEM((S, 1), jnp.float32),
        ],
    )(x, w_gate)


_ROWS_G = (B * K) // NW


@functools.cache
def _sc_mesh():
    return plsc.VectorSubcoreMesh(core_axis_name="c", subcore_axis_name="s",
                                  num_cores=NC, num_subcores=NS)


@functools.cache
def _gather_kernel():
    @functools.partial(
        pl.kernel,
        out_type=jax.ShapeDtypeStruct((B * K, D), jnp.float32),
        mesh=_sc_mesh(),
        compiler_params=pltpu.CompilerParams(needs_layout_passes=False),
        scratch_types=[
            pltpu.VMEM((_ROWS_G,), jnp.int32),
            pltpu.VMEM((_ROWS_G, D), jnp.float32),
            pltpu.SemaphoreType.DMA,
        ],
    )
    def gather(xf_hbm, gidx_hbm, sel_hbm, idx_v, rows_v, sem):
        wid = lax.axis_index("s") * NC + lax.axis_index("c")
        base = wid * _ROWS_G
        pltpu.sync_copy(gidx_hbm.at[pl.ds(base, _ROWS_G)], idx_v)
        pltpu.async_copy(xf_hbm.at[idx_v], rows_v, sem).wait()
        pltpu.sync_copy(rows_v, sel_hbm.at[pl.ds(base, _ROWS_G)])

    return gather


def _gather(xf, gidx):
    return _gather_kernel()(xf, gidx)


_CQ = 256


def _attn_body(sel_ref, th_ref, wq_ref, wk_ref, wv_ref, wo_ref, out_ref,
               o_scr):
    h = sel_ref[0]
    th = th_ref[...]
    cos = jnp.cos(th)
    sin = jnp.sin(th)
    hn = (h * lax.rsqrt(jnp.mean(h * h, axis=-1, keepdims=True) + 1e-6)
          ).astype(jnp.bfloat16)
    q = jnp.dot(hn, wq_ref[...].astype(jnp.bfloat16),
                preferred_element_type=jnp.float32)
    k = jnp.dot(hn, wk_ref[...].astype(jnp.bfloat16),
                preferred_element_type=jnp.float32)
    v = jnp.dot(hn, wv_ref[...].astype(jnp.bfloat16),
                preferred_element_type=jnp.float32).astype(jnp.bfloat16)
    ones1 = jnp.ones((K, 1), jnp.bfloat16)
    masks = []
    for ci in range(K // _CQ):
        kl = (ci + 1) * _CQ
        r = lax.broadcasted_iota(jnp.int32, (_CQ, kl), 0) + ci * _CQ
        c = lax.broadcasted_iota(jnp.int32, (_CQ, kl), 1)
        masks.append((r >= c).astype(jnp.float32))
    for hd in range(NH):
        o = hd * HD
        q1, q2 = q[:, o:o + 32], q[:, o + 32:o + 64]
        k1, k2 = k[:, o:o + 32], k[:, o + 32:o + 64]
        rq = (jnp.concatenate(
            [q1 * cos - q2 * sin, q2 * cos + q1 * sin], 1)
            * 0.125).astype(jnp.bfloat16)
        rk = jnp.concatenate([k1 * cos - k2 * sin, k2 * cos + k1 * sin],
                             1).astype(jnp.bfloat16)
        vp = jnp.concatenate([v[:, o:o + HD], ones1], axis=1)
        for ci in range(K // _CQ):
            kl = (ci + 1) * _CQ
            lo = lax.dot_general(rq[ci * _CQ:kl], rk[:kl],
                                 (((1,), (1,)), ((), ())),
                                 preferred_element_type=jnp.float32)
            p = (jnp.exp(lo) * masks[ci]).astype(jnp.bfloat16)
            ov = jnp.dot(p, vp[:kl], preferred_element_type=jnp.float32)
            o_scr[ci * _CQ:kl, o:o + HD] = (
                ov[:, :HD] * (1.0 / ov[:, HD:HD + 1])).astype(jnp.bfloat16)
    out_ref[0] = h + jnp.dot(o_scr[...], wo_ref[...].astype(jnp.bfloat16),
                             preferred_element_type=jnp.float32)


def _attn(selb, theta, wq, wk, wv, wo):
    return pl.pallas_call(
        _attn_body,
        grid=(B,),
        in_specs=[
            pl.BlockSpec((1, K, D), lambda b: (b, 0, 0)),
            pl.BlockSpec((K, HD // 2), lambda b: (0, 0)),
            pl.BlockSpec((D, D), lambda b: (0, 0)),
            pl.BlockSpec((D, D), lambda b: (0, 0)),
            pl.BlockSpec((D, D), lambda b: (0, 0)),
            pl.BlockSpec((D, D), lambda b: (0, 0)),
        ],
        out_specs=pl.BlockSpec((1, K, D), lambda b: (b, 0, 0)),
        out_shape=jax.ShapeDtypeStruct((B, K, D), jnp.float32),
        scratch_shapes=[pltpu.VMEM((K, D), jnp.bfloat16)],
    )(selb, theta, wq, wk, wv, wo)


_RCH = 256


def _mlp_body(h_ref, w1_ref, w2_ref, out_ref):
    h = h_ref[0]
    hn = (h * lax.rsqrt(jnp.mean(h * h, axis=-1, keepdims=True) + 1e-6)
          ).astype(jnp.bfloat16)
    a = jax.nn.gelu(jnp.dot(hn, w1_ref[...].astype(jnp.bfloat16),
                            preferred_element_type=jnp.float32))
    out_ref[0] = h + jnp.dot(a.astype(jnp.bfloat16),
                             w2_ref[...].astype(jnp.bfloat16),
                             preferred_element_type=jnp.float32)


def _mlp(h, w1, w2):
    return pl.pallas_call(
        _mlp_body,
        grid=(B, K // _RCH),
        in_specs=[
            pl.BlockSpec((1, _RCH, D), lambda b, r: (b, r, 0)),
            pl.BlockSpec((D, DFF), lambda b, r: (0, 0)),
            pl.BlockSpec((DFF, D), lambda b, r: (0, 0)),
        ],
        out_specs=pl.BlockSpec((1, _RCH, D), lambda b, r: (b, r, 0)),
        out_shape=jax.ShapeDtypeStruct((B, K, D), jnp.float32),
    )(h, w1, w2)


@functools.cache
def _combine_kernel():
    @functools.partial(
        pl.kernel,
        out_type=jax.ShapeDtypeStruct((B * S, D), jnp.float32),
        mesh=_sc_mesh(),
        compiler_params=pltpu.CompilerParams(needs_layout_passes=False),
        scratch_types=[
            pltpu.VMEM((_ROWS_G,), jnp.int32),
            pltpu.VMEM((_ROWS_G, D), jnp.float32),
            pltpu.SemaphoreType.DMA,
        ],
    )
    def combine(xf_hbm, bf_hbm, gidx_hbm, uidx_hbm, out_hbm,
                idx_v, buf, sem):
        wid = lax.axis_index("s") * NC + lax.axis_index("c")
        base = wid * _ROWS_G
        pltpu.sync_copy(uidx_hbm.at[pl.ds(base, _ROWS_G)], idx_v)
        pltpu.async_copy(xf_hbm.at[idx_v], buf, sem).wait()
        pltpu.async_copy(buf, out_hbm.at[idx_v], sem).wait()
        pltpu.sync_copy(bf_hbm.at[pl.ds(base, _ROWS_G)], buf)
        pltpu.sync_copy(gidx_hbm.at[pl.ds(base, _ROWS_G)], idx_v)
        pltpu.async_copy(buf, out_hbm.at[idx_v], sem).wait()

    return combine


def _combine(xf, bf, gidx, uidx):
    return _combine_kernel()(xf, bf, gidx, uidx)


def _rope_perm():
    one = np.concatenate([np.arange(0, HD, 2), np.arange(1, HD, 2)])
    return np.concatenate([one + HD * h for h in range(NH)])


_PERM = _rope_perm()


def kernel(x, freqs_cis, w_gate, Wq, Wk, Wv, Wo, W1, W2):
    wg2 = w_gate.reshape(1, D)
    gidx, uidx, aux = _route(x, wg2)

    x_flat = x.reshape(B * S, D)
    gidx_f = gidx.reshape(B * K)
    selected = _gather(x_flat, gidx_f)
    selb = selected.reshape(B, K, D)

    h1 = _attn(selb, freqs_cis[:K], Wq[:, _PERM], Wk[:, _PERM], Wv, Wo)
    block_out = _mlp(h1, W1, W2)

    out = _combine(x_flat, block_out.reshape(B * K, D),
                   gidx_f, uidx.reshape(B * K))
    return out.reshape(B, S, D), aux[0, 0]
